# baseline (speedup 1.0000x reference)
import functools

import jax
import jax.numpy as jnp
from jax import lax
from jax.experimental import pallas as pl
from jax.experimental.pallas import tpu as pltpu
from jax.experimental.pallas import tpu_sc as plsc


def _sc_pool_sum(idx, pair, table2, num_cores, num_subcores):
    B, R = idx.shape
    _, C, K = pair.shape
    _, DP = table2.shape
    D = DP // 2
    NV = D // 16
    NI = R // 16
    BPW = B // (num_cores * num_subcores)

    mesh = plsc.VectorSubcoreMesh(core_axis_name="c", subcore_axis_name="s")
    NROW = 2

    @functools.partial(
        pl.kernel,
        mesh=mesh,
        out_type=jax.ShapeDtypeStruct((B, D), jnp.float32),
        scratch_types=[
            pltpu.VMEM((NROW, R), jnp.int32),
            pltpu.VMEM((BPW, C, K), jnp.int32),
            [[pltpu.VMEM((K, DP), jnp.float32)] * C] * NROW,
            pltpu.VMEM((BPW, D), jnp.float32),
            [pltpu.SemaphoreType.DMA] * NROW,
        ],
        compiler_params=pltpu.CompilerParams(needs_layout_passes=False),
    )
    def pool(idx_hbm, pair_hbm, table_hbm, out_hbm, idx_v, pair_v, rows_v,
             acc_v, sems):
        wid = lax.axis_index("s") * num_cores + lax.axis_index("c")
        base = wid * BPW
        pltpu.sync_copy(pair_hbm.at[pl.ds(base, BPW)], pair_v)

        def issue(b, rslot):
            pltpu.make_async_copy(
                idx_hbm.at[base + b], idx_v.at[rslot], sems[rslot]
            ).start()
            for c in range(C):
                pltpu.make_async_copy(
                    table_hbm.at[pair_v.at[b, c]],
                    rows_v[rslot][c],
                    sems[rslot],
                ).start()

        def wait_row(rslot):
            for c in range(C):
                pltpu.make_async_copy(
                    table_hbm.at[pl.ds(0, K)], rows_v[rslot][c], sems[rslot]
                ).wait()
            pltpu.make_async_copy(
                idx_hbm.at[0], idx_v.at[rslot], sems[rslot]
            ).wait()

        for rslot in range(NROW):
            issue(rslot, rslot)

        zero = jnp.zeros((16,), jnp.float32)

        def group_body(g, carry):
            for rslot in range(NROW):
                b = g * NROW + rslot
                wait_row(rslot)

                accs = (zero,) * NV
                for k in range(R // 16):
                    offv = (idx_v[rslot, pl.ds(16 * k, 16)] & 1) * D
                    for j in range(16):
                        pos = 16 * k + j
                        sel, prow = divmod(pos, K)
                        off = offv[j]
                        accs = tuple(
                            accs[v] + rows_v[rslot][sel][
                                prow, pl.ds(off + 16 * v, 16)]
                            for v in range(NV)
                        )
                for v in range(NV):
                    acc_v[b, pl.ds(16 * v, 16)] = accs[v]

                nb = b + NROW

                @pl.when(nb < BPW)
                def _():
                    issue(nb, rslot)

            return carry

        lax.fori_loop(0, BPW // NROW, group_body, 0)
        pltpu.sync_copy(acc_v, out_hbm.at[pl.ds(base, BPW)])

    return pool(idx, pair, table2)


def _mask_correct_mlp(inputs, sums, table0, W1, b1, W2, b2, pad_per_row):
    B, L = inputs.shape
    D = sums.shape[1]
    H = W1.shape[1]
    BLK = 1024

    def body(inp_ref, sums_ref, t0_ref, W1_ref, b1_ref, W2_ref, b2_ref, out_ref):
        cnt = jnp.sum(
            (inp_ref[...] == 0).astype(jnp.float32), axis=1, keepdims=True
        )
        pooled = sums_ref[...] - (cnt + pad_per_row) * t0_ref[...]
        h = jnp.tanh(
            jnp.dot(pooled, W1_ref[...], preferred_element_type=jnp.float32)
            + b1_ref[...]
        )
        out_ref[...] = (
            jnp.dot(h, W2_ref[...], preferred_element_type=jnp.float32)
            + b2_ref[...]
        )

    return pl.pallas_call(
        body,
        grid=(B // BLK,),
        in_specs=[
            pl.BlockSpec((BLK, L), lambda i: (i, 0)),
            pl.BlockSpec((BLK, D), lambda i: (i, 0)),
            pl.BlockSpec((1, D), lambda i: (0, 0)),
            pl.BlockSpec((D, H), lambda i: (0, 0)),
            pl.BlockSpec((1, H), lambda i: (0, 0)),
            pl.BlockSpec((H, D), lambda i: (0, 0)),
            pl.BlockSpec((1, D), lambda i: (0, 0)),
        ],
        out_specs=pl.BlockSpec((BLK, D), lambda i: (i, 0)),
        out_shape=jax.ShapeDtypeStruct((B, D), jnp.float32),
    )(inputs, sums, table0, W1, b1, W2, b2)


def kernel(inputs, table, W1, b1, W2, b2):
    B, L = inputs.shape

    info = plsc.get_sparse_core_info()

    R = -(-L // 16) * 16
    pad = R - L
    idx = jnp.pad(inputs, ((0, 0), (0, pad)))

    pair = (idx >> 1).reshape(B, 2, R // 2)
    sums = _sc_pool_sum(
        idx, pair, table.reshape(-1, 2 * table.shape[1]),
        info.num_cores, info.num_subcores,
    )
    return _mask_correct_mlp(
        inputs,
        sums,
        table[0:1],
        W1,
        b1.reshape(1, -1),
        W2,
        b2.reshape(1, -1),
        float(pad),
    )


# speedup vs baseline: 1.0002x; 1.0002x over previous
import functools

import jax
import jax.numpy as jnp
from jax import lax
from jax.experimental import pallas as pl
from jax.experimental.pallas import tpu as pltpu
from jax.experimental.pallas import tpu_sc as plsc


def _sc_pool_sum(idx, pair, table2, num_cores, num_subcores):
    B, R = idx.shape
    C = pair.shape[0] // B
    K = pair.shape[2]
    _, DP = table2.shape
    D = DP // 2
    NV = D // 16
    NI = R // 16
    BPW = B // (num_cores * num_subcores)

    mesh = plsc.VectorSubcoreMesh(core_axis_name="c", subcore_axis_name="s")
    NROW = 2

    @functools.partial(
        pl.kernel,
        mesh=mesh,
        out_type=jax.ShapeDtypeStruct((B, D), jnp.float32),
        scratch_types=[
            pltpu.VMEM((NROW, R), jnp.int32),
            pltpu.VMEM((BPW * C, 1, K), jnp.int32),
            [[pltpu.VMEM((K, DP), jnp.float32)] * C] * NROW,
            pltpu.VMEM((BPW, D), jnp.float32),
            [pltpu.SemaphoreType.DMA] * NROW,
        ],
        compiler_params=pltpu.CompilerParams(needs_layout_passes=False),
    )
    def pool(idx_hbm, pair_hbm, table_hbm, out_hbm, idx_v, pair_v, rows_v,
             acc_v, sems):
        wid = lax.axis_index("s") * num_cores + lax.axis_index("c")
        base = wid * BPW
        pltpu.sync_copy(pair_hbm.at[pl.ds(base * C, BPW * C)], pair_v)

        def issue(b, rslot):
            pltpu.make_async_copy(
                idx_hbm.at[base + b], idx_v.at[rslot], sems[rslot]
            ).start()
            for c in range(C):
                pltpu.make_async_copy(
                    table_hbm.at[pair_v.at[b * C + c, 0]],
                    rows_v[rslot][c],
                    sems[rslot],
                ).start()

        def wait_row(rslot):
            for c in range(C):
                pltpu.make_async_copy(
                    table_hbm.at[pl.ds(0, K)], rows_v[rslot][c], sems[rslot]
                ).wait()
            pltpu.make_async_copy(
                idx_hbm.at[0], idx_v.at[rslot], sems[rslot]
            ).wait()

        for rslot in range(NROW):
            issue(rslot, rslot)

        zero = jnp.zeros((16,), jnp.float32)

        def group_body(g, carry):
            for rslot in range(NROW):
                b = g * NROW + rslot
                wait_row(rslot)

                accs = (zero,) * NV
                for k in range(R // 16):
                    offv = (idx_v[rslot, pl.ds(16 * k, 16)] & 1) * D
                    for j in range(16):
                        pos = 16 * k + j
                        sel, prow = divmod(pos, K)
                        off = offv[j]
                        accs = tuple(
                            accs[v] + rows_v[rslot][sel][
                                prow, pl.ds(off + 16 * v, 16)]
                            for v in range(NV)
                        )
                for v in range(NV):
                    acc_v[b, pl.ds(16 * v, 16)] = accs[v]

                nb = b + NROW

                @pl.when(nb < BPW)
                def _():
                    issue(nb, rslot)

            return carry

        lax.fori_loop(0, BPW // NROW, group_body, 0)
        pltpu.sync_copy(acc_v, out_hbm.at[pl.ds(base, BPW)])

    return pool(idx, pair, table2)


def _mask_correct_mlp(inputs, sums, table0, W1, b1, W2, b2, pad_per_row):
    B, L = inputs.shape
    D = sums.shape[1]
    H = W1.shape[1]
    BLK = 1024

    def body(inp_ref, sums_ref, t0_ref, W1_ref, b1_ref, W2_ref, b2_ref, out_ref):
        cnt = jnp.sum(
            (inp_ref[...] == 0).astype(jnp.float32), axis=1, keepdims=True
        )
        pooled = sums_ref[...] - (cnt + pad_per_row) * t0_ref[...]
        h = jnp.tanh(
            jnp.dot(pooled, W1_ref[...], preferred_element_type=jnp.float32)
            + b1_ref[...]
        )
        out_ref[...] = (
            jnp.dot(h, W2_ref[...], preferred_element_type=jnp.float32)
            + b2_ref[...]
        )

    return pl.pallas_call(
        body,
        grid=(B // BLK,),
        in_specs=[
            pl.BlockSpec((BLK, L), lambda i: (i, 0)),
            pl.BlockSpec((BLK, D), lambda i: (i, 0)),
            pl.BlockSpec((1, D), lambda i: (0, 0)),
            pl.BlockSpec((D, H), lambda i: (0, 0)),
            pl.BlockSpec((1, H), lambda i: (0, 0)),
            pl.BlockSpec((H, D), lambda i: (0, 0)),
            pl.BlockSpec((1, D), lambda i: (0, 0)),
        ],
        out_specs=pl.BlockSpec((BLK, D), lambda i: (i, 0)),
        out_shape=jax.ShapeDtypeStruct((B, D), jnp.float32),
    )(inputs, sums, table0, W1, b1, W2, b2)


def kernel(inputs, table, W1, b1, W2, b2):
    B, L = inputs.shape

    info = plsc.get_sparse_core_info()

    R = -(-L // 16) * 16
    pad = R - L
    idx = jnp.pad(inputs, ((0, 0), (0, pad)))

    pair = (idx >> 1).reshape(B * 2, 1, R // 2)
    sums = _sc_pool_sum(
        idx, pair, table.reshape(-1, 2 * table.shape[1]),
        info.num_cores, info.num_subcores,
    )
    return _mask_correct_mlp(
        inputs,
        sums,
        table[0:1],
        W1,
        b1.reshape(1, -1),
        W2,
        b2.reshape(1, -1),
        float(pad),
    )


# speedup vs baseline: 1.0010x; 1.0008x over previous
import functools

import jax
import jax.numpy as jnp
from jax import lax
from jax.experimental import pallas as pl
from jax.experimental.pallas import tpu as pltpu
from jax.experimental.pallas import tpu_sc as plsc


def _sc_pool_sum(idx, pair, table2, num_cores, num_subcores):
    B, R = idx.shape
    C = pair.shape[0] // B
    K = pair.shape[2]
    _, DP = table2.shape
    D = DP // 2
    NV = D // 16
    NI = R // 16
    BPW = B // (num_cores * num_subcores)

    mesh = plsc.VectorSubcoreMesh(core_axis_name="c", subcore_axis_name="s")
    NROW = 2

    @functools.partial(
        pl.kernel,
        mesh=mesh,
        out_type=jax.ShapeDtypeStruct((B, D), jnp.float32),
        scratch_types=[
            pltpu.VMEM((BPW, R), jnp.int32),
            pltpu.VMEM((BPW * C, 1, K), jnp.int32),
            [[pltpu.VMEM((K, DP), jnp.float32)] * C] * NROW,
            pltpu.VMEM((NROW, D), jnp.float32),
            [[pltpu.SemaphoreType.DMA] * C] * NROW,
            [pltpu.SemaphoreType.DMA] * NROW,
        ],
        compiler_params=pltpu.CompilerParams(needs_layout_passes=False),
    )
    def pool(idx_hbm, pair_hbm, table_hbm, out_hbm, idx_v, pair_v, rows_v,
             acc_v, sems, osems):
        wid = lax.axis_index("s") * num_cores + lax.axis_index("c")
        base = wid * BPW
        pltpu.sync_copy(idx_hbm.at[pl.ds(base, BPW)], idx_v)
        pltpu.sync_copy(pair_hbm.at[pl.ds(base * C, BPW * C)], pair_v)

        def issue(b, rslot):
            for c in range(C):
                pltpu.make_async_copy(
                    table_hbm.at[pair_v.at[b * C + c, 0]],
                    rows_v[rslot][c],
                    sems[rslot][c],
                ).start()

        def wait_row(rslot):
            for c in range(C):
                pltpu.make_async_copy(
                    table_hbm.at[pl.ds(0, K)], rows_v[rslot][c],
                    sems[rslot][c],
                ).wait()

        for rslot in range(NROW):
            issue(rslot, rslot)

        zero = jnp.zeros((16,), jnp.float32)

        def group_body(g, carry):
            for rslot in range(NROW):
                b = g * NROW + rslot
                wait_row(rslot)

                def acc_lo(k, acc):
                    offv = (idx_v[b, pl.ds(16 * k, 16)] & 1) * D
                    for j in range(16):
                        off = offv[j]
                        acc = tuple(
                            acc[v] + rows_v[rslot][0][
                                16 * k + j, pl.ds(off + 16 * v, 16)]
                            for v in range(NV)
                        )
                    return acc

                def acc_hi(k, acc):
                    offv = (idx_v[b, pl.ds(K + 8 + 16 * k, 16)] & 1) * D
                    for j in range(16):
                        off = offv[j]
                        acc = tuple(
                            acc[v] + rows_v[rslot][1][
                                8 + 16 * k + j, pl.ds(off + 16 * v, 16)]
                            for v in range(NV)
                        )
                    return acc

                accs = lax.fori_loop(0, K // 16, acc_lo, (zero,) * NV)
                KB = (K // 16) * 16
                offv = (idx_v[b, pl.ds(KB, 16)] & 1) * D
                for j in range(16):
                    pos = KB + j
                    sel, prow = divmod(pos, K)
                    off = offv[j]
                    accs = tuple(
                        accs[v] + rows_v[rslot][sel][
                            prow, pl.ds(off + 16 * v, 16)]
                        for v in range(NV)
                    )
                accs = lax.fori_loop(0, (R - KB - 16) // 16, acc_hi, accs)

                @pl.when(b >= NROW)
                def _():
                    pltpu.make_async_copy(
                        out_hbm.at[0], acc_v.at[rslot], osems[rslot]
                    ).wait()

                for v in range(NV):
                    acc_v[rslot, pl.ds(16 * v, 16)] = accs[v]
                pltpu.make_async_copy(
                    acc_v.at[rslot], out_hbm.at[base + b], osems[rslot]
                ).start()

                nb = b + NROW

                @pl.when(nb < BPW)
                def _():
                    issue(nb, rslot)

            return carry

        lax.fori_loop(0, BPW // NROW, group_body, 0)
        for rslot in range(NROW):
            pltpu.make_async_copy(
                out_hbm.at[0], acc_v.at[rslot], osems[rslot]
            ).wait()

    return pool(idx, pair, table2)


def _mask_correct_mlp(inputs, sums, table0, W1, b1, W2, b2, pad_per_row):
    B, L = inputs.shape
    D = sums.shape[1]
    H = W1.shape[1]
    BLK = 1024

    def body(inp_ref, sums_ref, t0_ref, W1_ref, b1_ref, W2_ref, b2_ref, out_ref):
        cnt = jnp.sum(
            (inp_ref[...] == 0).astype(jnp.float32), axis=1, keepdims=True
        )
        pooled = sums_ref[...] - (cnt + pad_per_row) * t0_ref[...]
        h = jnp.tanh(
            jnp.dot(pooled, W1_ref[...], preferred_element_type=jnp.float32)
            + b1_ref[...]
        )
        out_ref[...] = (
            jnp.dot(h, W2_ref[...], preferred_element_type=jnp.float32)
            + b2_ref[...]
        )

    return pl.pallas_call(
        body,
        grid=(B // BLK,),
        in_specs=[
            pl.BlockSpec((BLK, L), lambda i: (i, 0)),
            pl.BlockSpec((BLK, D), lambda i: (i, 0)),
            pl.BlockSpec((1, D), lambda i: (0, 0)),
            pl.BlockSpec((D, H), lambda i: (0, 0)),
            pl.BlockSpec((1, H), lambda i: (0, 0)),
            pl.BlockSpec((H, D), lambda i: (0, 0)),
            pl.BlockSpec((1, D), lambda i: (0, 0)),
        ],
        out_specs=pl.BlockSpec((BLK, D), lambda i: (i, 0)),
        out_shape=jax.ShapeDtypeStruct((B, D), jnp.float32),
    )(inputs, sums, table0, W1, b1, W2, b2)


def kernel(inputs, table, W1, b1, W2, b2):
    B, L = inputs.shape

    info = plsc.get_sparse_core_info()

    R = -(-L // 16) * 16
    pad = R - L
    idx = jnp.pad(inputs, ((0, 0), (0, pad)))

    pair = (idx >> 1).reshape(B * 2, 1, R // 2)
    sums = _sc_pool_sum(
        idx, pair, table.reshape(-1, 2 * table.shape[1]),
        info.num_cores, info.num_subcores,
    )
    return _mask_correct_mlp(
        inputs,
        sums,
        table[0:1],
        W1,
        b1.reshape(1, -1),
        W2,
        b2.reshape(1, -1),
        float(pad),
    )


# speedup vs baseline: 1.5339x; 1.5323x over previous
import functools

import jax
import jax.numpy as jnp
from jax import lax
from jax.experimental import pallas as pl
from jax.experimental.pallas import tpu as pltpu
from jax.experimental.pallas import tpu_sc as plsc


def _sc_pool_sum(idx, table, num_cores, num_subcores):
    B, C, K = idx.shape
    V, D = table.shape
    NV = D // 16
    BPW = B // (num_cores * num_subcores)
    R = C * K

    mesh = plsc.VectorSubcoreMesh(core_axis_name="c", subcore_axis_name="s")
    NBUF = 4

    @functools.partial(
        pl.kernel,
        mesh=mesh,
        out_type=jax.ShapeDtypeStruct((B, D), jnp.float32),
        scratch_types=[
            pltpu.VMEM((BPW, C, K), jnp.int32),
            pltpu.VMEM((NBUF, R, D), jnp.float32),
            pltpu.VMEM((BPW, D), jnp.float32),
            [pltpu.SemaphoreType.DMA] * NBUF,
        ],
        compiler_params=pltpu.CompilerParams(use_tc_tiling_on_sc=False),
    )
    def pool(idx_hbm, table_hbm, out_hbm, idx_v, rows_v, acc_v, sems):
        wid = lax.axis_index("s") * num_cores + lax.axis_index("c")
        base = wid * BPW
        pltpu.sync_copy(idx_hbm.at[pl.ds(base, BPW)], idx_v)

        def issue(b, slot):
            for c in range(C):
                pltpu.make_async_copy(
                    table_hbm.at[idx_v.at[b, c]],
                    rows_v.at[slot, pl.ds(c * K, K)],
                    sems[slot],
                ).start()

        def wait_slot(slot):
            pltpu.make_async_copy(
                table_hbm.at[pl.ds(0, R)], rows_v.at[slot], sems[slot]
            ).wait()

        for slot in range(NBUF):
            issue(slot, slot)

        zero = jnp.zeros((16,), jnp.float32)

        def group_body(g, carry):
            for slot in range(NBUF):
                b = g * NBUF + slot
                wait_slot(slot)

                def acc_body(r, acc):
                    return tuple(
                        acc[v] + rows_v[slot, r, pl.ds(16 * v, 16)]
                        for v in range(NV)
                    )

                acc = lax.fori_loop(0, R, acc_body, (zero,) * NV, unroll=8)
                for v in range(NV):
                    acc_v[b, pl.ds(16 * v, 16)] = acc[v]

                nb = b + NBUF

                @pl.when(nb < BPW)
                def _():
                    issue(nb, slot)

            return carry

        lax.fori_loop(0, BPW // NBUF, group_body, 0)
        pltpu.sync_copy(acc_v, out_hbm.at[pl.ds(base, BPW)])

    return pool(idx, table)


def _mask_correct_mlp(inputs, sums, table0, W1, b1, W2, b2, pad_per_row):
    B, L = inputs.shape
    D = sums.shape[1]
    H = W1.shape[1]
    BLK = 1024

    def body(inp_ref, sums_ref, t0_ref, W1_ref, b1_ref, W2_ref, b2_ref, out_ref):
        cnt = jnp.sum(
            (inp_ref[...] == 0).astype(jnp.float32), axis=1, keepdims=True
        )
        pooled = sums_ref[...] - (cnt + pad_per_row) * t0_ref[...]
        h = jnp.tanh(
            jnp.dot(pooled, W1_ref[...], preferred_element_type=jnp.float32)
            + b1_ref[...]
        )
        out_ref[...] = (
            jnp.dot(h, W2_ref[...], preferred_element_type=jnp.float32)
            + b2_ref[...]
        )

    return pl.pallas_call(
        body,
        grid=(B // BLK,),
        in_specs=[
            pl.BlockSpec((BLK, L), lambda i: (i, 0)),
            pl.BlockSpec((BLK, D), lambda i: (i, 0)),
            pl.BlockSpec((1, D), lambda i: (0, 0)),
            pl.BlockSpec((D, H), lambda i: (0, 0)),
            pl.BlockSpec((1, H), lambda i: (0, 0)),
            pl.BlockSpec((H, D), lambda i: (0, 0)),
            pl.BlockSpec((1, D), lambda i: (0, 0)),
        ],
        out_specs=pl.BlockSpec((BLK, D), lambda i: (i, 0)),
        out_shape=jax.ShapeDtypeStruct((B, D), jnp.float32),
    )(inputs, sums, table0, W1, b1, W2, b2)


def kernel(inputs, table, W1, b1, W2, b2):
    B, L = inputs.shape
    info = plsc.get_sparse_core_info()

    K = 104
    C = -(-L // K)
    pad = C * K - L
    idx = jnp.pad(inputs, ((0, 0), (0, pad))).reshape(B, C, K)

    sums = _sc_pool_sum(idx, table, info.num_cores, info.num_subcores)
    return _mask_correct_mlp(
        inputs,
        sums,
        table[0:1],
        W1,
        b1.reshape(1, -1),
        W2,
        b2.reshape(1, -1),
        float(pad),
    )
